# baseline (speedup 1.0000x reference)
import jax
import jax.numpy as jnp
from jax.experimental import pallas as pl
from jax.experimental.pallas import tpu as pltpu

_BM = 256


def _ff_body(x_ref, w_ref, o_ref):
    o_ref[...] = jax.lax.dot(
        x_ref[...].astype(jnp.bfloat16),
        w_ref[...].astype(jnp.bfloat16),
        preferred_element_type=jnp.float32,
    ).astype(jnp.bfloat16)


def _feature_transform(x, w):
    n = x.shape[0]
    d_out = w.shape[1]
    return pl.pallas_call(
        _ff_body,
        out_shape=jax.ShapeDtypeStruct((n, d_out), jnp.bfloat16),
    )(x, w)


def _agg_body(s_ref, b_ref, adj_ref, o_ref):
    a = adj_ref[...].astype(jnp.bfloat16)
    y = jax.lax.dot(a, s_ref[...], preferred_element_type=jnp.float32)
    y = y + b_ref[...]
    o_ref[...] = jnp.where(y >= 0, y, 0.01 * y)


def _aggregate(adj, support, b):
    n = adj.shape[0]
    d = support.shape[1]
    return pl.pallas_call(
        _agg_body,
        grid=(n // _BM,),
        in_specs=[
            pl.BlockSpec((n, d), lambda i: (0, 0)),
            pl.BlockSpec((1, d), lambda i: (0, 0)),
            pl.BlockSpec((_BM, n), lambda i: (i, 0)),
        ],
        out_specs=pl.BlockSpec((_BM, d), lambda i: (i, 0)),
        out_shape=jax.ShapeDtypeStruct((n, d), jnp.float32),
        compiler_params=pltpu.CompilerParams(
            dimension_semantics=("arbitrary",),
        ),
    )(support, b.reshape(1, d), adj)


def kernel(x, adj, W1, b1, W2, b2):
    s1 = _feature_transform(x, W1)
    h = _aggregate(adj, s1, b1)
    s2 = _feature_transform(h, W2)
    return _aggregate(adj, s2, b2)
